# baseline (speedup 1.0000x reference)
import functools

import jax
import jax.numpy as jnp
from jax import lax
from jax.experimental import pallas as pl
from jax.experimental.pallas import tpu as pltpu
from jax.experimental.pallas import tpu_sc as plsc

L = 16
ROWS = 24576
W = 2048
NC = 2
NS = 16
NW = NC * NS
RPW = ROWS // NW
CO = 64
NSL = RPW // CO
CL = 32
CV = W // L
HL = 128


def _scalar_max(v):
    m = v[0]
    for l in range(1, L):
        m = jnp.maximum(m, v[l])
    return m


def _row_lane_max(buf):
    def scan(j, bv):
        return jnp.maximum(bv, buf[0, pl.ds(j * L, L)])

    return lax.fori_loop(1, CV, scan, buf[0, pl.ds(0, L)])


def _slow_row(full_buf, patch_buf, state, csum_ref, thr, iota):
    state[1] = 0
    state[2] = 0
    csum_ref[0] = jnp.float32(0.0)

    def find_body(_, carry):
        @pl.when(state[2] == 0)
        def _():
            p = state[1]
            csum = csum_ref[0]

            bv = full_buf[0, pl.ds(0, L)]
            bi = iota

            def scan(j, bvbi):
                bv, bi = bvbi
                v = full_buf[0, pl.ds(j * L, L)]
                upd = v > bv
                return (jnp.where(upd, v, bv),
                        jnp.where(upd, iota + j * L, bi))

            bv, bi = lax.fori_loop(1, CV, scan, (bv, bi))

            m, idx = bv[0], bi[0]
            for l in range(1, L):
                v, i = bv[l], bi[l]
                better = jnp.logical_or(
                    v > m, jnp.logical_and(v == m, i < idx))
                m = jnp.where(better, v, m)
                idx = jnp.where(better, i, idx)

            csum2 = csum + m
            sel = csum2 <= thr

            @pl.when(sel)
            def _():
                pbase = (p // L) * L
                oc = patch_buf[0, pl.ds(pbase, L)]
                patch_buf[0, pl.ds(pbase, L)] = jnp.where(
                    iota == p - pbase, idx, oc)
                ibase = (idx // L) * L
                vc = full_buf[0, pl.ds(ibase, L)]
                full_buf[0, pl.ds(ibase, L)] = jnp.where(
                    iota == idx - ibase, jnp.float32(-1.0), vc)
                state[1] = p + 1
                csum_ref[0] = csum2

            @pl.when(jnp.logical_or(jnp.logical_not(sel), p + 1 >= W))
            def _():
                state[2] = 1

        return carry

    lax.fori_loop(0, W, find_body, 0)


@functools.partial(
    pl.kernel,
    out_type=jax.ShapeDtypeStruct((ROWS, W), jnp.int32),
    mesh=plsc.VectorSubcoreMesh(core_axis_name="c", subcore_axis_name="s"),
    scratch_types=[
        pltpu.VMEM((CO, HL), jnp.float32),
        pltpu.VMEM((CO, HL), jnp.float32),
        pltpu.VMEM((CL, W), jnp.int32),
        pltpu.VMEM_SHARED((CO, W), jnp.int32),
        pltpu.VMEM((1, W), jnp.int32),
        pltpu.VMEM((1, W), jnp.float32),
        pltpu.VMEM((L,), jnp.float32),
        pltpu.SMEM((4,), jnp.int32),
        pltpu.SMEM((1,), jnp.float32),
        pltpu.SemaphoreType.DMA,
        pltpu.SemaphoreType.DMA,
        pltpu.SemaphoreType.DMA,
    ],
)
def _topp_kernel(atn_hbm, sig_hbm, out_hbm,
                 hdr0, hdr1, neg1_buf, shared_neg1, patch_buf, full_buf,
                 sig_buf, state, csum_ref, in_sem0, in_sem1, out_sem):
    c = lax.axis_index("c")
    s = lax.axis_index("s")
    wid = s * NC + c
    base_row = wid * RPW

    iota = lax.iota(jnp.int32, L)
    neg1v = jnp.full((L,), -1, jnp.int32)

    def rows_of(g):
        return pl.ds(pl.multiple_of(base_row + g * CO, CO), CO)

    def hdr_src(g):
        return atn_hbm.at[rows_of(g), pl.ds(0, HL)]

    pltpu.sync_copy(sig_hbm, sig_buf)
    sigv = sig_buf[...]
    sig_scalars = [sigv[h] for h in range(12)]

    for r in range(CL):
        def fill(j, _, r=r):
            neg1_buf[r, pl.ds(j * L, L)] = neg1v
            return 0

        lax.fori_loop(0, CV, fill, 0)

    pltpu.sync_copy(neg1_buf, shared_neg1.at[pl.ds(0, CL)])
    pltpu.sync_copy(neg1_buf, shared_neg1.at[pl.ds(CL, CL)])
    plsc.subcore_barrier()

    def pfill(j, _):
        patch_buf[0, pl.ds(j * L, L)] = neg1v
        return 0

    lax.fori_loop(0, CV, pfill, 0)

    def drain_one_out(_, carry):
        pltpu.make_async_copy(
            shared_neg1, out_hbm.at[rows_of(0)], out_sem).wait()
        return carry

    def handle_suspect(row, thr):
        pltpu.sync_copy(atn_hbm.at[pl.ds(row, 1)], full_buf)
        m = _scalar_max(_row_lane_max(full_buf))

        @pl.when(m <= thr)
        def _():
            lax.fori_loop(0, state[3], drain_one_out, 0)
            state[3] = 0
            _slow_row(full_buf, patch_buf, state, csum_ref, thr, iota)
            pltpu.sync_copy(patch_buf, out_hbm.at[pl.ds(row, 1)])

            def refill(j, _):
                patch_buf[0, pl.ds(j * L, L)] = neg1v
                return 0

            lax.fori_loop(0, CV, refill, 0)

    def process(g, hdr_buf):
        row0 = base_row + g * CO

        head = row0 >> 11
        thr = sig_scalars[11]
        for h in range(10, -1, -1):
            thr = jnp.where(head == h, sig_scalars[h], thr)
        pltpu.async_copy(shared_neg1, out_hbm.at[rows_of(g)], out_sem)
        state[3] = state[3] + 1

        @pl.when(state[3] > 5)
        def _():
            lax.fori_loop(0, 1, drain_one_out, 0)
            state[3] = state[3] - 1

        def classify(r, ns):
            hv = hdr_buf[r, pl.ds(0, L)]
            for j in range(1, HL // L):
                hv = jnp.maximum(hv, hdr_buf[r, pl.ds(j * L, L)])
            return ns + jnp.where(_scalar_max(hv) <= thr, 1, 0)

        nsus = lax.fori_loop(0, CO, classify, 0)

        @pl.when(nsus > 0)
        def _():
            def handle(r, carry):
                hv = hdr_buf[r, pl.ds(0, L)]
                for j in range(1, HL // L):
                    hv = jnp.maximum(hv, hdr_buf[r, pl.ds(j * L, L)])

                @pl.when(_scalar_max(hv) <= thr)
                def _():
                    handle_suspect(row0 + r, thr)

                return carry

            lax.fori_loop(0, CO, handle, 0)

    state[3] = 0

    pltpu.async_copy(hdr_src(0), hdr0, in_sem0)

    def outer(gg, carry):
        g0 = gg * 2
        g1 = g0 + 1
        pltpu.async_copy(hdr_src(g1), hdr1, in_sem1)
        pltpu.make_async_copy(hdr_src(g0), hdr0, in_sem0).wait()
        process(g0, hdr0)
        @pl.when(g0 + 2 < NSL)
        def _():
            pltpu.async_copy(hdr_src(g0 + 2), hdr0, in_sem0)

        pltpu.make_async_copy(hdr_src(g1), hdr1, in_sem1).wait()
        process(g1, hdr1)
        return carry

    lax.fori_loop(0, NSL // 2, outer, 0)

    lax.fori_loop(0, state[3], drain_one_out, 0)


def kernel(atn, threshold):
    batch, num_heads, seq, seq2 = atn.shape
    atn2 = atn.reshape(num_heads * seq, seq2)
    sig = jax.nn.sigmoid(threshold.astype(jnp.float32))
    sig16 = jnp.pad(sig, (0, L - sig.shape[0]))
    out = _topp_kernel(atn2, sig16)
    return out.reshape(batch, num_heads * seq, seq2)


# speedup vs baseline: 1.3091x; 1.3091x over previous
import functools

import jax
import jax.numpy as jnp
from jax import lax
from jax.experimental import pallas as pl
from jax.experimental.pallas import tpu as pltpu
from jax.experimental.pallas import tpu_sc as plsc

L = 16
ROWS = 24576
W = 2048
NC = 2
NS = 16
NW = NC * NS
RPW = ROWS // NW
CO = 32
NSL = RPW // CO
CV = W // L
HL = 128


def _scalar_max(v):
    m = v[0]
    for l in range(1, L):
        m = jnp.maximum(m, v[l])
    return m


def _row_lane_max(buf):
    def scan(j, bv):
        return jnp.maximum(bv, buf[0, pl.ds(j * L, L)])

    return lax.fori_loop(1, CV, scan, buf[0, pl.ds(0, L)])


def _slow_row(full_buf, patch_buf, state, csum_ref, thr, iota):
    state[1] = 0
    state[2] = 0
    csum_ref[0] = jnp.float32(0.0)

    def find_body(_, carry):
        @pl.when(state[2] == 0)
        def _():
            p = state[1]
            csum = csum_ref[0]

            bv = full_buf[0, pl.ds(0, L)]
            bi = iota

            def scan(j, bvbi):
                bv, bi = bvbi
                v = full_buf[0, pl.ds(j * L, L)]
                upd = v > bv
                return (jnp.where(upd, v, bv),
                        jnp.where(upd, iota + j * L, bi))

            bv, bi = lax.fori_loop(1, CV, scan, (bv, bi))

            m, idx = bv[0], bi[0]
            for l in range(1, L):
                v, i = bv[l], bi[l]
                better = jnp.logical_or(
                    v > m, jnp.logical_and(v == m, i < idx))
                m = jnp.where(better, v, m)
                idx = jnp.where(better, i, idx)

            csum2 = csum + m
            sel = csum2 <= thr

            @pl.when(sel)
            def _():
                pbase = (p // L) * L
                oc = patch_buf[0, pl.ds(pbase, L)]
                patch_buf[0, pl.ds(pbase, L)] = jnp.where(
                    iota == p - pbase, idx, oc)
                ibase = (idx // L) * L
                vc = full_buf[0, pl.ds(ibase, L)]
                full_buf[0, pl.ds(ibase, L)] = jnp.where(
                    iota == idx - ibase, jnp.float32(-1.0), vc)
                state[1] = p + 1
                csum_ref[0] = csum2

            @pl.when(jnp.logical_or(jnp.logical_not(sel), p + 1 >= W))
            def _():
                state[2] = 1

        return carry

    lax.fori_loop(0, W, find_body, 0)


@functools.partial(
    pl.kernel,
    out_type=jax.ShapeDtypeStruct((ROWS, W), jnp.int32),
    mesh=plsc.VectorSubcoreMesh(core_axis_name="c", subcore_axis_name="s"),
    scratch_types=[
        pltpu.VMEM((CO, HL), jnp.float32),
        pltpu.VMEM((CO, HL), jnp.float32),
        pltpu.VMEM((CO, W), jnp.int32),
        pltpu.VMEM_SHARED((CO, W), jnp.int32),
        pltpu.VMEM((1, W), jnp.int32),
        pltpu.VMEM((1, W), jnp.float32),
        pltpu.VMEM((L,), jnp.float32),
        pltpu.SMEM((8,), jnp.int32),
        pltpu.SMEM((1,), jnp.float32),
        pltpu.SemaphoreType.DMA,
        pltpu.SemaphoreType.DMA,
        pltpu.SemaphoreType.DMA,
        pltpu.SemaphoreType.DMA,
    ],
)
def _topp_kernel(atn_hbm, sig_hbm, out_hbm,
                 hdr0, hdr1, neg1_buf, shared_neg1, patch_buf, full_buf,
                 sig_buf, state, csum_ref, in_sem0, in_sem1, out_sem,
                 out_sem2):
    c = lax.axis_index("c")
    s = lax.axis_index("s")
    wid = s * NC + c
    base_row = wid * RPW

    iota = lax.iota(jnp.int32, L)
    neg1v = jnp.full((L,), -1, jnp.int32)

    def rows_of(g):
        return pl.ds(pl.multiple_of(base_row + g * CO, CO), CO)

    def hdr_src(g):
        return atn_hbm.at[rows_of(g), pl.ds(0, HL)]

    pltpu.sync_copy(sig_hbm, sig_buf)
    sigv = sig_buf[...]
    sig_scalars = [sigv[h] for h in range(12)]

    for r in range(CO):
        def fill(j, _, r=r):
            neg1_buf[r, pl.ds(j * L, L)] = neg1v
            return 0

        lax.fori_loop(0, CV, fill, 0)

    pltpu.sync_copy(neg1_buf, shared_neg1)
    plsc.subcore_barrier()

    def pfill(j, _):
        patch_buf[0, pl.ds(j * L, L)] = neg1v
        return 0

    lax.fori_loop(0, CV, pfill, 0)

    def drain_one_out(_, carry):
        pltpu.make_async_copy(
            neg1_buf, out_hbm.at[rows_of(0)], out_sem).wait()
        return carry

    def drain_one_out2(_, carry):
        pltpu.make_async_copy(
            shared_neg1, out_hbm.at[rows_of(0)], out_sem2).wait()
        return carry

    def drain_all(_unused=None):
        lax.fori_loop(0, state[3], drain_one_out, 0)
        state[3] = 0
        lax.fori_loop(0, state[5], drain_one_out2, 0)
        state[5] = 0

    def handle_suspect(row, thr):
        pltpu.sync_copy(atn_hbm.at[pl.ds(row, 1)], full_buf)
        m = _scalar_max(_row_lane_max(full_buf))

        @pl.when(m <= thr)
        def _():
            drain_all()
            _slow_row(full_buf, patch_buf, state, csum_ref, thr, iota)
            pltpu.sync_copy(patch_buf, out_hbm.at[pl.ds(row, 1)])

            def refill(j, _):
                patch_buf[0, pl.ds(j * L, L)] = neg1v
                return 0

            lax.fori_loop(0, CV, refill, 0)

    def process(g, hdr_buf):
        row0 = base_row + g * CO

        head = row0 >> 11
        thr = sig_scalars[11]
        for h in range(10, -1, -1):
            thr = jnp.where(head == h, sig_scalars[h], thr)
        use_local = (g % 8) < 5

        @pl.when(use_local)
        def _():
            pltpu.async_copy(neg1_buf, out_hbm.at[rows_of(g)], out_sem)
            state[3] = state[3] + 1

            @pl.when(state[3] > 3)
            def _():
                lax.fori_loop(0, 1, drain_one_out, 0)
                state[3] = state[3] - 1

        @pl.when(jnp.logical_not(use_local))
        def _():
            pltpu.async_copy(shared_neg1, out_hbm.at[rows_of(g)], out_sem2)
            state[5] = state[5] + 1

            @pl.when(state[5] > 3)
            def _():
                lax.fori_loop(0, 1, drain_one_out2, 0)
                state[5] = state[5] - 1

        def classify(r, ns):
            hv = hdr_buf[r, pl.ds(0, L)]
            for j in range(1, HL // L):
                hv = jnp.maximum(hv, hdr_buf[r, pl.ds(j * L, L)])
            return ns + jnp.where(_scalar_max(hv) <= thr, 1, 0)

        nsus = lax.fori_loop(0, CO, classify, 0)

        @pl.when(nsus > 0)
        def _():
            def handle(r, carry):
                hv = hdr_buf[r, pl.ds(0, L)]
                for j in range(1, HL // L):
                    hv = jnp.maximum(hv, hdr_buf[r, pl.ds(j * L, L)])

                @pl.when(_scalar_max(hv) <= thr)
                def _():
                    handle_suspect(row0 + r, thr)

                return carry

            lax.fori_loop(0, CO, handle, 0)

    state[3] = 0
    state[5] = 0

    pltpu.async_copy(hdr_src(0), hdr0, in_sem0)

    def outer(gg, carry):
        g0 = gg * 2
        g1 = g0 + 1
        pltpu.async_copy(hdr_src(g1), hdr1, in_sem1)
        pltpu.make_async_copy(hdr_src(g0), hdr0, in_sem0).wait()
        process(g0, hdr0)
        @pl.when(g0 + 2 < NSL)
        def _():
            pltpu.async_copy(hdr_src(g0 + 2), hdr0, in_sem0)

        pltpu.make_async_copy(hdr_src(g1), hdr1, in_sem1).wait()
        process(g1, hdr1)
        return carry

    lax.fori_loop(0, NSL // 2, outer, 0)

    lax.fori_loop(0, state[3], drain_one_out, 0)
    lax.fori_loop(0, state[5], drain_one_out2, 0)


def kernel(atn, threshold):
    batch, num_heads, seq, seq2 = atn.shape
    atn2 = atn.reshape(num_heads * seq, seq2)
    sig = jax.nn.sigmoid(threshold.astype(jnp.float32))
    sig16 = jnp.pad(sig, (0, L - sig.shape[0]))
    out = _topp_kernel(atn2, sig16)
    return out.reshape(batch, num_heads * seq, seq2)
